# Initial kernel scaffold; baseline (speedup 1.0000x reference)
import jax
import jax.numpy as jnp
from jax.experimental import pallas as pl


def kernel(x, vec, edge_index, r_ij, f_ij, d_ij, ln_w, ln_b, Wq, bq, Wk, bk, Wv, bv, Wo, bo, Wvec, Wdk, bdk, Wdv, bdv):
    raise NotImplementedError("write your pallas kernel here")


# speedup vs baseline: 23.0620x; 23.0620x over previous
import functools
import math

import numpy as np
import jax
import jax.numpy as jnp
from jax import lax
from jax.experimental import pallas as pl
from jax.experimental.pallas import tpu as pltpu
from jax.experimental.pallas import tpu_sc as plsc

N = 10000
E = 320000
HC = 128
NH = 8
HD = 16
NRBF = 64
CUTOFF = 5.0
EPS = 1e-5
NPAD = 10240

KVW = 7 * HC

BN = 1000
BE = 1280

NW = 32
EW = E // NW
GCH = 80
ET = E // 16


def _silu(x):
    return x * jax.nn.sigmoid(x)


def _node_prep_body(x_ref, vec_ref, lnw_ref, lnb_ref, wqt_ref, bq_ref,
                    wkt_ref, bk_ref, wvt_ref, bv_ref, wvect_ref,
                    q_ref, kvv_ref, vp_ref):
    x = x_ref[...]
    mu = jnp.mean(x, axis=-1, keepdims=True)
    xc = x - mu
    var = jnp.mean(xc * xc, axis=-1, keepdims=True)
    xn = xc * lax.rsqrt(var + EPS) * lnw_ref[...] + lnb_ref[...]
    q_ref[...] = jnp.dot(xn, wqt_ref[...], preferred_element_type=jnp.float32) + bq_ref[...]
    kvv_ref[:, 0:HC] = jnp.dot(xn, wkt_ref[...], preferred_element_type=jnp.float32) + bk_ref[...]
    kvv_ref[:, HC:4 * HC] = jnp.dot(xn, wvt_ref[...], preferred_element_type=jnp.float32) + bv_ref[...]
    for d in range(3):
        vslice = vec_ref[:, d, :]
        kvv_ref[:, (4 + d) * HC:(5 + d) * HC] = vslice
        vp_ref[:, d, :] = jnp.dot(vslice, wvect_ref[...], preferred_element_type=jnp.float32)


def _node_prep(x, vec, lnw, lnb, wqt, bq, wkt, bk, wvt, bv, wvect):
    grid = (N // BN,)
    return pl.pallas_call(
        _node_prep_body,
        grid=grid,
        in_specs=[
            pl.BlockSpec((BN, HC), lambda i: (i, 0)),
            pl.BlockSpec((BN, 3, HC), lambda i: (i, 0, 0)),
            pl.BlockSpec((1, HC), lambda i: (0, 0)),
            pl.BlockSpec((1, HC), lambda i: (0, 0)),
            pl.BlockSpec((HC, HC), lambda i: (0, 0)),
            pl.BlockSpec((1, HC), lambda i: (0, 0)),
            pl.BlockSpec((HC, HC), lambda i: (0, 0)),
            pl.BlockSpec((1, HC), lambda i: (0, 0)),
            pl.BlockSpec((HC, 3 * HC), lambda i: (0, 0)),
            pl.BlockSpec((1, 3 * HC), lambda i: (0, 0)),
            pl.BlockSpec((HC, 3 * HC), lambda i: (0, 0)),
        ],
        out_specs=[
            pl.BlockSpec((BN, HC), lambda i: (i, 0)),
            pl.BlockSpec((BN, KVW), lambda i: (i, 0)),
            pl.BlockSpec((BN, 3, 3 * HC), lambda i: (i, 0, 0)),
        ],
        out_shape=[
            jax.ShapeDtypeStruct((N, HC), jnp.float32),
            jax.ShapeDtypeStruct((N, KVW), jnp.float32),
            jax.ShapeDtypeStruct((N, 3, 3 * HC), jnp.float32),
        ],
    )(x, vec, lnw, lnb, wqt, bq, wkt, bk, wvt, bv, wvect)


def _sc_gather(dst, src, q, kvv):
    mesh = plsc.VectorSubcoreMesh(core_axis_name="c", subcore_axis_name="s")

    @functools.partial(
        pl.kernel,
        out_type=[
            jax.ShapeDtypeStruct((E, HC), jnp.float32),
            jax.ShapeDtypeStruct((E, KVW), jnp.float32),
        ],
        mesh=mesh,
        scratch_types=[
            pltpu.VMEM((GCH,), jnp.int32),
            pltpu.VMEM((GCH,), jnp.int32),
            pltpu.VMEM((GCH, HC), jnp.float32),
            pltpu.VMEM((GCH, KVW), jnp.float32),
            pltpu.SemaphoreType.DMA,
            pltpu.SemaphoreType.DMA,
        ],
    )
    def gather_k(dst_hbm, src_hbm, q_hbm, kvv_hbm, qi_hbm, kvj_hbm,
                 dstv, srcv, qbuf, kvbuf, sem1, sem2):
        wid = lax.axis_index("s") * 2 + lax.axis_index("c")

        def chunk(i, carry):
            base = wid * EW + i * GCH
            pltpu.sync_copy(dst_hbm.at[pl.ds(base, GCH)], dstv)
            pltpu.sync_copy(src_hbm.at[pl.ds(base, GCH)], srcv)
            cp1 = pltpu.async_copy(q_hbm.at[dstv], qbuf, sem1)
            cp2 = pltpu.async_copy(kvv_hbm.at[srcv], kvbuf, sem2)
            cp1.wait()
            cp2.wait()
            pltpu.sync_copy(qbuf, qi_hbm.at[pl.ds(base, GCH)])
            pltpu.sync_copy(kvbuf, kvj_hbm.at[pl.ds(base, GCH)])
            return carry

        lax.fori_loop(0, EW // GCH, chunk, 0)

    return gather_k(dst, src, q, kvv)


def _edge_body(f_ref, qi_ref, kvj_ref, r_ref, dT_ref,
               wdkt_ref, bdk_ref, wdvt_ref, bdv_ref, m_ref, y_ref):
    f = f_ref[...]
    dk = _silu(jnp.dot(f, wdkt_ref[...], preferred_element_type=jnp.float32) + bdk_ref[...])
    dv = _silu(jnp.dot(f, wdvt_ref[...], preferred_element_type=jnp.float32) + bdv_ref[...])
    kj = kvj_ref[:, 0:HC]
    vj = kvj_ref[:, HC:4 * HC]
    t = qi_ref[...] * kj * dk
    attnb = jnp.dot(t, m_ref[...], preferred_element_type=jnp.float32)
    r = r_ref[...]
    cut = 0.5 * (jnp.cos(r * (math.pi / CUTOFF)) + 1.0)
    cut = jnp.where(r < CUTOFF, cut, 0.0)
    act = _silu(attnb) * cut
    vdv = vj * dv
    v1 = vdv[:, 0:HC]
    v2 = vdv[:, HC:2 * HC]
    v3 = vdv[:, 2 * HC:3 * HC]
    y_ref[0] = v1 * act
    for d in range(3):
        y_ref[1 + d] = kvj_ref[:, (4 + d) * HC:(5 + d) * HC] * v2 + v3 * dT_ref[d]


def _edge_compute(f_ij, qi, kvj, r2, dT, wdkt, bdk, wdvt, bdv, m):
    grid = (E // BE,)
    return pl.pallas_call(
        _edge_body,
        grid=grid,
        in_specs=[
            pl.BlockSpec((BE, NRBF), lambda i: (i, 0)),
            pl.BlockSpec((BE, HC), lambda i: (i, 0)),
            pl.BlockSpec((BE, KVW), lambda i: (i, 0)),
            pl.BlockSpec((BE, 1), lambda i: (i, 0)),
            pl.BlockSpec((3, BE, 1), lambda i: (0, i, 0)),
            pl.BlockSpec((NRBF, HC), lambda i: (0, 0)),
            pl.BlockSpec((1, HC), lambda i: (0, 0)),
            pl.BlockSpec((NRBF, 3 * HC), lambda i: (0, 0)),
            pl.BlockSpec((1, 3 * HC), lambda i: (0, 0)),
            pl.BlockSpec((HC, HC), lambda i: (0, 0)),
        ],
        out_specs=pl.BlockSpec((4, BE, HC), lambda i: (0, i, 0)),
        out_shape=jax.ShapeDtypeStruct((4, E, HC), jnp.float32),
    )(f_ij, qi, kvj, r2, dT, wdkt, bdk, wdvt, bdv, m)


NHALF = NPAD // 2
ACC_R = 5248
ZSTRIPE = ACC_R // 16
FSTRIPE = NHALF // 16
BCH = 160
IB = 80


def _sc_scatter(y, dst):
    mesh = plsc.VectorSubcoreMesh(core_axis_name="c", subcore_axis_name="s")

    @functools.partial(
        pl.kernel,
        out_type=jax.ShapeDtypeStruct((4, NPAD, HC), jnp.float32),
        mesh=mesh,
        scratch_types=[
            pltpu.VMEM_SHARED((ACC_R, HC), jnp.float32),
            pltpu.VMEM((ZSTRIPE, HC), jnp.float32),
            pltpu.VMEM((BCH, HC), jnp.float32),
            pltpu.VMEM((BCH,), jnp.int32),
            pltpu.VMEM((BCH // IB, IB), jnp.int32),
        ],
    )
    def scatter_k(y_hbm, dst_hbm, agg_hbm, acc, zbuf, vbuf, idxv, idx2v):
        c = lax.axis_index("c")
        t = lax.axis_index("s")
        half_base = c * NHALF
        zero16 = jnp.zeros((16,), jnp.float32)

        def zrow(rr, carry):
            for j in range(HC // 16):
                zbuf[rr, pl.ds(j * 16, 16)] = zero16
            return carry

        lax.fori_loop(0, ZSTRIPE, zrow, 0)
        pltpu.sync_copy(zbuf, acc.at[pl.ds(t * ZSTRIPE, ZSTRIPE)])
        plsc.subcore_barrier()

        for slab in range(4):
            def chunk(i, carry):
                base = t * ET + i * BCH
                pltpu.sync_copy(dst_hbm.at[pl.ds(base, BCH)], idxv)
                pltpu.sync_copy(y_hbm.at[slab, pl.ds(base, BCH)], vbuf)
                for g in range(BCH // 16):
                    v = idxv[pl.ds(g * 16, 16)]
                    local = v - half_base
                    keep = (local >= 0) & (local < NHALF)
                    gj, gk = divmod(g, IB // 16)
                    idx2v[gj, pl.ds(gk * 16, 16)] = jnp.where(keep, local, NHALF)
                for j in range(BCH // IB):
                    pltpu.sync_copy(vbuf.at[pl.ds(j * IB, IB)],
                                    acc.at[idx2v.at[j]], add=True)
                return carry

            lax.fori_loop(0, ET // BCH, chunk, 0)
            plsc.subcore_barrier()
            pltpu.sync_copy(
                acc.at[pl.ds(t * FSTRIPE, FSTRIPE)],
                agg_hbm.at[slab, pl.ds(pl.multiple_of(half_base + t * FSTRIPE, 8),
                                       FSTRIPE)])
            if slab < 3:
                pltpu.sync_copy(zbuf, acc.at[pl.ds(t * ZSTRIPE, ZSTRIPE)])
            plsc.subcore_barrier()

    return scatter_k(y, dst)


def _final_body(agg_ref, vp_ref, wot_ref, bo_ref, dx_ref, dvec_ref):
    o = jnp.dot(agg_ref[0], wot_ref[...], preferred_element_type=jnp.float32) + bo_ref[...]
    o1 = o[:, 0:HC]
    o2 = o[:, HC:2 * HC]
    o3 = o[:, 2 * HC:3 * HC]
    vec_dot = (vp_ref[:, 0, 0:HC] * vp_ref[:, 0, HC:2 * HC]
               + vp_ref[:, 1, 0:HC] * vp_ref[:, 1, HC:2 * HC]
               + vp_ref[:, 2, 0:HC] * vp_ref[:, 2, HC:2 * HC])
    dx_ref[...] = vec_dot * o2 + o3
    for d in range(3):
        dvec_ref[:, d, :] = vp_ref[:, d, 2 * HC:3 * HC] * o1 + agg_ref[1 + d]


def _final(agg, vp, wot, bo):
    grid = (N // BN,)
    return pl.pallas_call(
        _final_body,
        grid=grid,
        in_specs=[
            pl.BlockSpec((4, BN, HC), lambda i: (0, i, 0)),
            pl.BlockSpec((BN, 3, 3 * HC), lambda i: (i, 0, 0)),
            pl.BlockSpec((HC, 3 * HC), lambda i: (0, 0)),
            pl.BlockSpec((1, 3 * HC), lambda i: (0, 0)),
        ],
        out_specs=[
            pl.BlockSpec((BN, HC), lambda i: (i, 0)),
            pl.BlockSpec((BN, 3, HC), lambda i: (i, 0, 0)),
        ],
        out_shape=[
            jax.ShapeDtypeStruct((N, HC), jnp.float32),
            jax.ShapeDtypeStruct((N, 3, HC), jnp.float32),
        ],
    )(agg, vp, wot, bo)


_s, _h, _j = np.meshgrid(np.arange(3), np.arange(NH), np.arange(HD), indexing="ij")
_PERM = (_h * 3 * HD + _s * HD + _j).reshape(-1)
_M_HEAD = np.kron(np.eye(NH, dtype=np.float32),
                  np.ones((HD, HD), np.float32))


def kernel(x, vec, edge_index, r_ij, f_ij, d_ij, ln_w, ln_b, Wq, bq, Wk, bk,
           Wv, bv, Wo, bo, Wvec, Wdk, bdk, Wdv, bdv):
    src = edge_index[0]
    dst = edge_index[1]

    wqt = Wq.T
    wkt = Wk.T
    wvt = Wv[_PERM].T
    bv_p = bv[_PERM].reshape(1, 3 * HC)
    wvect = Wvec.T
    wdkt = Wdk.T
    wdvt = Wdv[_PERM].T
    bdv_p = bdv[_PERM].reshape(1, 3 * HC)
    wot = Wo.T

    q, kvv, vp = _node_prep(
        x, vec, ln_w.reshape(1, HC), ln_b.reshape(1, HC),
        wqt, bq.reshape(1, HC), wkt, bk.reshape(1, HC), wvt, bv_p, wvect)

    qi, kvj = _sc_gather(dst, src, q, kvv)

    r2 = r_ij.reshape(E, 1)
    dT = d_ij.T.reshape(3, E, 1)
    y = _edge_compute(f_ij, qi, kvj, r2, dT, wdkt,
                      bdk.reshape(1, HC), wdvt, bdv_p, _M_HEAD)

    agg = _sc_scatter(y, dst)

    dx, dvec = _final(agg, vp, wot, bo.reshape(1, 3 * HC))
    return dx, dvec


# speedup vs baseline: 24.8591x; 1.0779x over previous
import functools
import math

import numpy as np
import jax
import jax.numpy as jnp
from jax import lax
from jax.experimental import pallas as pl
from jax.experimental.pallas import tpu as pltpu
from jax.experimental.pallas import tpu_sc as plsc

N = 10000
E = 320000
HC = 128
NH = 8
HD = 16
NRBF = 64
CUTOFF = 5.0
EPS = 1e-5
NPAD = 10240

KVW = 7 * HC

BN = 1000
BE = 1280

NW = 32
EW = E // NW
GCH = 40
ET = E // 16


def _silu(x):
    return x * jax.nn.sigmoid(x)


def _node_prep_body(x_ref, vec_ref, lnw_ref, lnb_ref, wqt_ref, bq_ref,
                    wkt_ref, bk_ref, wvt_ref, bv_ref, wvect_ref,
                    q_ref, kvv_ref, vp_ref):
    x = x_ref[...]
    mu = jnp.mean(x, axis=-1, keepdims=True)
    xc = x - mu
    var = jnp.mean(xc * xc, axis=-1, keepdims=True)
    xn = xc * lax.rsqrt(var + EPS) * lnw_ref[...] + lnb_ref[...]
    q_ref[...] = jnp.dot(xn, wqt_ref[...], preferred_element_type=jnp.float32) + bq_ref[...]
    kvv_ref[:, 0:HC] = jnp.dot(xn, wkt_ref[...], preferred_element_type=jnp.float32) + bk_ref[...]
    kvv_ref[:, HC:4 * HC] = jnp.dot(xn, wvt_ref[...], preferred_element_type=jnp.float32) + bv_ref[...]
    for d in range(3):
        vslice = vec_ref[:, d, :]
        kvv_ref[:, (4 + d) * HC:(5 + d) * HC] = vslice
        vp_ref[:, d, :] = jnp.dot(vslice, wvect_ref[...], preferred_element_type=jnp.float32)


def _node_prep(x, vec, lnw, lnb, wqt, bq, wkt, bk, wvt, bv, wvect):
    grid = (N // BN,)
    return pl.pallas_call(
        _node_prep_body,
        grid=grid,
        in_specs=[
            pl.BlockSpec((BN, HC), lambda i: (i, 0)),
            pl.BlockSpec((BN, 3, HC), lambda i: (i, 0, 0)),
            pl.BlockSpec((1, HC), lambda i: (0, 0)),
            pl.BlockSpec((1, HC), lambda i: (0, 0)),
            pl.BlockSpec((HC, HC), lambda i: (0, 0)),
            pl.BlockSpec((1, HC), lambda i: (0, 0)),
            pl.BlockSpec((HC, HC), lambda i: (0, 0)),
            pl.BlockSpec((1, HC), lambda i: (0, 0)),
            pl.BlockSpec((HC, 3 * HC), lambda i: (0, 0)),
            pl.BlockSpec((1, 3 * HC), lambda i: (0, 0)),
            pl.BlockSpec((HC, 3 * HC), lambda i: (0, 0)),
        ],
        out_specs=[
            pl.BlockSpec((BN, HC), lambda i: (i, 0)),
            pl.BlockSpec((BN, KVW), lambda i: (i, 0)),
            pl.BlockSpec((BN, 3, 3 * HC), lambda i: (i, 0, 0)),
        ],
        out_shape=[
            jax.ShapeDtypeStruct((N, HC), jnp.float32),
            jax.ShapeDtypeStruct((N, KVW), jnp.float32),
            jax.ShapeDtypeStruct((N, 3, 3 * HC), jnp.float32),
        ],
    )(x, vec, lnw, lnb, wqt, bq, wkt, bk, wvt, bv, wvect)


def _sc_gather(dst, src, q, kvv):
    mesh = plsc.VectorSubcoreMesh(core_axis_name="c", subcore_axis_name="s")

    @functools.partial(
        pl.kernel,
        out_type=[
            jax.ShapeDtypeStruct((E, HC), jnp.float32),
            jax.ShapeDtypeStruct((E, KVW), jnp.float32),
        ],
        mesh=mesh,
        scratch_types=[
            pltpu.VMEM((EW,), jnp.int32),
            pltpu.VMEM((EW,), jnp.int32),
            pltpu.VMEM((GCH, HC), jnp.float32),
            pltpu.VMEM((GCH, HC), jnp.float32),
            pltpu.VMEM((GCH, KVW), jnp.float32),
            pltpu.VMEM((GCH, KVW), jnp.float32),
            pltpu.SemaphoreType.DMA,
            pltpu.SemaphoreType.DMA,
            pltpu.SemaphoreType.DMA,
            pltpu.SemaphoreType.DMA,
            pltpu.SemaphoreType.DMA,
            pltpu.SemaphoreType.DMA,
        ],
    )
    def gather_k(dst_hbm, src_hbm, q_hbm, kvv_hbm, qi_hbm, kvj_hbm,
                 dstv, srcv, qbuf0, qbuf1, kvbuf0, kvbuf1,
                 qs0, qs1, ks0, ks1, ws0, ws1):
        wid = lax.axis_index("s") * 2 + lax.axis_index("c")
        wbase = wid * EW
        qbufs = (qbuf0, qbuf1)
        kvbufs = (kvbuf0, kvbuf1)
        qsems = (qs0, qs1)
        ksems = (ks0, ks1)
        wsems = (ws0, ws1)

        pltpu.sync_copy(dst_hbm.at[pl.ds(wbase, EW)], dstv)
        pltpu.sync_copy(src_hbm.at[pl.ds(wbase, EW)], srcv)

        @pl.loop(0, EW // GCH, step=2)
        def chunks(i):
            gq, gk = [], []
            for b in range(2):
                off = (i + b) * GCH
                gq.append(pltpu.async_copy(
                    q_hbm.at[dstv.at[pl.ds(off, GCH)]], qbufs[b], qsems[b]))
                gk.append(pltpu.async_copy(
                    kvv_hbm.at[srcv.at[pl.ds(off, GCH)]], kvbufs[b], ksems[b]))
            ws = []
            for b in range(2):
                off = (i + b) * GCH
                gq[b].wait()
                gk[b].wait()
                ws.append(pltpu.async_copy(
                    qbufs[b], qi_hbm.at[pl.ds(wbase + off, GCH)], wsems[b]))
                ws.append(pltpu.async_copy(
                    kvbufs[b], kvj_hbm.at[pl.ds(wbase + off, GCH)], wsems[b]))
            for w in ws:
                w.wait()

    return gather_k(dst, src, q, kvv)


def _edge_body(f_ref, qi_ref, kvj_ref, r_ref, dT_ref,
               wdkt_ref, bdk_ref, wdvt_ref, bdv_ref, m_ref, y_ref):
    f = f_ref[...]
    dk = _silu(jnp.dot(f, wdkt_ref[...], preferred_element_type=jnp.float32) + bdk_ref[...])
    dv = _silu(jnp.dot(f, wdvt_ref[...], preferred_element_type=jnp.float32) + bdv_ref[...])
    kj = kvj_ref[:, 0:HC]
    vj = kvj_ref[:, HC:4 * HC]
    t = qi_ref[...] * kj * dk
    attnb = jnp.dot(t, m_ref[...], preferred_element_type=jnp.float32)
    r = r_ref[...]
    cut = 0.5 * (jnp.cos(r * (math.pi / CUTOFF)) + 1.0)
    cut = jnp.where(r < CUTOFF, cut, 0.0)
    act = _silu(attnb) * cut
    vdv = vj * dv
    v1 = vdv[:, 0:HC]
    v2 = vdv[:, HC:2 * HC]
    v3 = vdv[:, 2 * HC:3 * HC]
    y_ref[0] = v1 * act
    for d in range(3):
        y_ref[1 + d] = kvj_ref[:, (4 + d) * HC:(5 + d) * HC] * v2 + v3 * dT_ref[d]


def _edge_compute(f_ij, qi, kvj, r2, dT, wdkt, bdk, wdvt, bdv, m):
    grid = (E // BE,)
    return pl.pallas_call(
        _edge_body,
        grid=grid,
        in_specs=[
            pl.BlockSpec((BE, NRBF), lambda i: (i, 0)),
            pl.BlockSpec((BE, HC), lambda i: (i, 0)),
            pl.BlockSpec((BE, KVW), lambda i: (i, 0)),
            pl.BlockSpec((BE, 1), lambda i: (i, 0)),
            pl.BlockSpec((3, BE, 1), lambda i: (0, i, 0)),
            pl.BlockSpec((NRBF, HC), lambda i: (0, 0)),
            pl.BlockSpec((1, HC), lambda i: (0, 0)),
            pl.BlockSpec((NRBF, 3 * HC), lambda i: (0, 0)),
            pl.BlockSpec((1, 3 * HC), lambda i: (0, 0)),
            pl.BlockSpec((HC, HC), lambda i: (0, 0)),
        ],
        out_specs=pl.BlockSpec((4, BE, HC), lambda i: (0, i, 0)),
        out_shape=jax.ShapeDtypeStruct((4, E, HC), jnp.float32),
    )(f_ij, qi, kvj, r2, dT, wdkt, bdk, wdvt, bdv, m)


NHALF = NPAD // 2
ACC_R = 5248
ZSTRIPE = ACC_R // 16
FSTRIPE = NHALF // 16
BCH = 80
IB = 80


def _sc_scatter(y, dst):
    mesh = plsc.VectorSubcoreMesh(core_axis_name="c", subcore_axis_name="s")

    @functools.partial(
        pl.kernel,
        out_type=jax.ShapeDtypeStruct((4, NPAD, HC), jnp.float32),
        mesh=mesh,
        scratch_types=[
            pltpu.VMEM_SHARED((ACC_R, HC), jnp.float32),
            pltpu.VMEM((BCH, HC), jnp.float32),
            pltpu.VMEM((BCH, HC), jnp.float32),
            pltpu.VMEM((BCH,), jnp.int32),
            pltpu.VMEM((BCH,), jnp.int32),
            pltpu.VMEM((BCH // IB, IB), jnp.int32),
            pltpu.VMEM((BCH // IB, IB), jnp.int32),
            pltpu.SemaphoreType.DMA,
            pltpu.SemaphoreType.DMA,
        ],
    )
    def scatter_k(y_hbm, dst_hbm, agg_hbm, acc,
                  vbuf0, vbuf1, idxv0, idxv1, idx2v0, idx2v1, sem0, sem1):
        c = lax.axis_index("c")
        t = lax.axis_index("s")
        half_base = c * NHALF
        zero16 = jnp.zeros((16,), jnp.float32)
        vbufs = (vbuf0, vbuf1)
        idxvs = (idxv0, idxv1)
        idx2vs = (idx2v0, idx2v1)
        sems = (sem0, sem1)

        def zero_acc():
            def zrow(rr, carry):
                for j in range(HC // 16):
                    vbuf0[rr, pl.ds(j * 16, 16)] = zero16
                return carry

            lax.fori_loop(0, BCH, zrow, 0)
            done = 0
            while done < ZSTRIPE:
                n = min(BCH, ZSTRIPE - done)
                pltpu.sync_copy(vbuf0.at[pl.ds(0, n)],
                                acc.at[pl.ds(t * ZSTRIPE + done, n)])
                done += n

        def remap(b):
            for g in range(BCH // 16):
                v = idxvs[b][pl.ds(g * 16, 16)]
                local = v - half_base
                keep = (local >= 0) & (local < NHALF)
                gj, gk = divmod(g, IB // 16)
                idx2vs[b][gj, pl.ds(gk * 16, 16)] = jnp.where(
                    keep, local, NHALF + (v & 127))

        def scatter_buf(b):
            for j in range(BCH // IB):
                pltpu.sync_copy(vbufs[b].at[pl.ds(j * IB, IB)],
                                acc.at[idx2vs[b].at[j]], add=True)

        zero_acc()
        plsc.subcore_barrier()

        for slab in range(4):
            @pl.loop(0, ET // BCH, step=2)
            def chunks(i):
                cps = []
                for b in range(2):
                    base = t * ET + (i + b) * BCH
                    pltpu.sync_copy(dst_hbm.at[pl.ds(base, BCH)], idxvs[b])
                    cps.append(pltpu.async_copy(y_hbm.at[slab, pl.ds(base, BCH)],
                                                vbufs[b], sems[b]))
                remap(0)
                remap(1)
                cps[0].wait()
                scatter_buf(0)
                cps[1].wait()
                scatter_buf(1)

            plsc.subcore_barrier()
            pltpu.sync_copy(
                acc.at[pl.ds(t * FSTRIPE, FSTRIPE)],
                agg_hbm.at[slab, pl.ds(pl.multiple_of(half_base + t * FSTRIPE, 8),
                                       FSTRIPE)])
            if slab < 3:
                zero_acc()
            plsc.subcore_barrier()

    return scatter_k(y, dst)


def _final_body(agg_ref, vp_ref, wot_ref, bo_ref, dx_ref, dvec_ref):
    o = jnp.dot(agg_ref[0], wot_ref[...], preferred_element_type=jnp.float32) + bo_ref[...]
    o1 = o[:, 0:HC]
    o2 = o[:, HC:2 * HC]
    o3 = o[:, 2 * HC:3 * HC]
    vec_dot = (vp_ref[:, 0, 0:HC] * vp_ref[:, 0, HC:2 * HC]
               + vp_ref[:, 1, 0:HC] * vp_ref[:, 1, HC:2 * HC]
               + vp_ref[:, 2, 0:HC] * vp_ref[:, 2, HC:2 * HC])
    dx_ref[...] = vec_dot * o2 + o3
    for d in range(3):
        dvec_ref[:, d, :] = vp_ref[:, d, 2 * HC:3 * HC] * o1 + agg_ref[1 + d]


def _final(agg, vp, wot, bo):
    grid = (N // BN,)
    return pl.pallas_call(
        _final_body,
        grid=grid,
        in_specs=[
            pl.BlockSpec((4, BN, HC), lambda i: (0, i, 0)),
            pl.BlockSpec((BN, 3, 3 * HC), lambda i: (i, 0, 0)),
            pl.BlockSpec((HC, 3 * HC), lambda i: (0, 0)),
            pl.BlockSpec((1, 3 * HC), lambda i: (0, 0)),
        ],
        out_specs=[
            pl.BlockSpec((BN, HC), lambda i: (i, 0)),
            pl.BlockSpec((BN, 3, HC), lambda i: (i, 0, 0)),
        ],
        out_shape=[
            jax.ShapeDtypeStruct((N, HC), jnp.float32),
            jax.ShapeDtypeStruct((N, 3, HC), jnp.float32),
        ],
    )(agg, vp, wot, bo)


_s, _h, _j = np.meshgrid(np.arange(3), np.arange(NH), np.arange(HD), indexing="ij")
_PERM = (_h * 3 * HD + _s * HD + _j).reshape(-1)
_M_HEAD = np.kron(np.eye(NH, dtype=np.float32),
                  np.ones((HD, HD), np.float32))


def kernel(x, vec, edge_index, r_ij, f_ij, d_ij, ln_w, ln_b, Wq, bq, Wk, bk,
           Wv, bv, Wo, bo, Wvec, Wdk, bdk, Wdv, bdv):
    src = edge_index[0]
    dst = edge_index[1]

    wqt = Wq.T
    wkt = Wk.T
    wvt = Wv[_PERM].T
    bv_p = bv[_PERM].reshape(1, 3 * HC)
    wvect = Wvec.T
    wdkt = Wdk.T
    wdvt = Wdv[_PERM].T
    bdv_p = bdv[_PERM].reshape(1, 3 * HC)
    wot = Wo.T

    q, kvv, vp = _node_prep(
        x, vec, ln_w.reshape(1, HC), ln_b.reshape(1, HC),
        wqt, bq.reshape(1, HC), wkt, bk.reshape(1, HC), wvt, bv_p, wvect)

    qi, kvj = _sc_gather(dst, src, q, kvv)

    r2 = r_ij.reshape(E, 1)
    dT = d_ij.T.reshape(3, E, 1)
    y = _edge_compute(f_ij, qi, kvj, r2, dT, wdkt,
                      bdk.reshape(1, HC), wdvt, bdv_p, _M_HEAD)

    agg = _sc_scatter(y, dst)

    dx, dvec = _final(agg, vp, wot, bo.reshape(1, 3 * HC))
    return dx, dvec


# speedup vs baseline: 27.0208x; 1.0870x over previous
import functools
import math

import numpy as np
import jax
import jax.numpy as jnp
from jax import lax
from jax.experimental import pallas as pl
from jax.experimental.pallas import tpu as pltpu
from jax.experimental.pallas import tpu_sc as plsc

N = 10000
E = 320000
HC = 128
NH = 8
HD = 16
NRBF = 64
CUTOFF = 5.0
EPS = 1e-5
NPAD = 10240

KVW = 7 * HC

BN = 1000
BE = 1280

NW = 32
EW = E // NW
GCH = 40
ET = E // 16


def _silu(x):
    return x * jax.nn.sigmoid(x)


def _node_prep_body(x_ref, vec_ref, lnw_ref, lnb_ref, wqt_ref, bq_ref,
                    wkt_ref, bk_ref, wvt_ref, bv_ref, wvect_ref,
                    q_ref, kvv_ref, vp_ref):
    x = x_ref[...]
    mu = jnp.mean(x, axis=-1, keepdims=True)
    xc = x - mu
    var = jnp.mean(xc * xc, axis=-1, keepdims=True)
    xn = xc * lax.rsqrt(var + EPS) * lnw_ref[...] + lnb_ref[...]
    q_ref[...] = jnp.dot(xn, wqt_ref[...], preferred_element_type=jnp.float32) + bq_ref[...]
    kvv_ref[:, 0:HC] = jnp.dot(xn, wkt_ref[...], preferred_element_type=jnp.float32) + bk_ref[...]
    kvv_ref[:, HC:4 * HC] = jnp.dot(xn, wvt_ref[...], preferred_element_type=jnp.float32) + bv_ref[...]
    for d in range(3):
        vslice = vec_ref[:, d, :]
        kvv_ref[:, (4 + d) * HC:(5 + d) * HC] = vslice
        vp_ref[:, d, :] = jnp.dot(vslice, wvect_ref[...], preferred_element_type=jnp.float32)


def _node_prep(x, vec, lnw, lnb, wqt, bq, wkt, bk, wvt, bv, wvect):
    grid = (N // BN,)
    return pl.pallas_call(
        _node_prep_body,
        grid=grid,
        in_specs=[
            pl.BlockSpec((BN, HC), lambda i: (i, 0)),
            pl.BlockSpec((BN, 3, HC), lambda i: (i, 0, 0)),
            pl.BlockSpec((1, HC), lambda i: (0, 0)),
            pl.BlockSpec((1, HC), lambda i: (0, 0)),
            pl.BlockSpec((HC, HC), lambda i: (0, 0)),
            pl.BlockSpec((1, HC), lambda i: (0, 0)),
            pl.BlockSpec((HC, HC), lambda i: (0, 0)),
            pl.BlockSpec((1, HC), lambda i: (0, 0)),
            pl.BlockSpec((HC, 3 * HC), lambda i: (0, 0)),
            pl.BlockSpec((1, 3 * HC), lambda i: (0, 0)),
            pl.BlockSpec((HC, 3 * HC), lambda i: (0, 0)),
        ],
        out_specs=[
            pl.BlockSpec((BN, HC), lambda i: (i, 0)),
            pl.BlockSpec((BN, KVW), lambda i: (i, 0)),
            pl.BlockSpec((BN, 3, 3 * HC), lambda i: (i, 0, 0)),
        ],
        out_shape=[
            jax.ShapeDtypeStruct((N, HC), jnp.float32),
            jax.ShapeDtypeStruct((N, KVW), jnp.float32),
            jax.ShapeDtypeStruct((N, 3, 3 * HC), jnp.float32),
        ],
    )(x, vec, lnw, lnb, wqt, bq, wkt, bk, wvt, bv, wvect)


def _sc_gather(dst, src, q, kvv, ne):
    mesh = plsc.VectorSubcoreMesh(core_axis_name="c", subcore_axis_name="s")
    ew = ne // NW
    nch = ew // GCH
    even_n = nch - (nch % 2)

    @functools.partial(
        pl.kernel,
        out_type=[
            jax.ShapeDtypeStruct((ne, HC), jnp.float32),
            jax.ShapeDtypeStruct((ne, KVW), jnp.float32),
        ],
        mesh=mesh,
        scratch_types=[
            pltpu.VMEM((ew,), jnp.int32),
            pltpu.VMEM((ew,), jnp.int32),
            pltpu.VMEM((GCH, HC), jnp.float32),
            pltpu.VMEM((GCH, HC), jnp.float32),
            pltpu.VMEM((GCH, KVW), jnp.float32),
            pltpu.VMEM((GCH, KVW), jnp.float32),
            pltpu.SemaphoreType.DMA,
            pltpu.SemaphoreType.DMA,
            pltpu.SemaphoreType.DMA,
            pltpu.SemaphoreType.DMA,
            pltpu.SemaphoreType.DMA,
            pltpu.SemaphoreType.DMA,
        ],
    )
    def gather_k(dst_hbm, src_hbm, q_hbm, kvv_hbm, qi_hbm, kvj_hbm,
                 dstv, srcv, qbuf0, qbuf1, kvbuf0, kvbuf1,
                 qs0, qs1, ks0, ks1, ws0, ws1):
        wid = lax.axis_index("s") * 2 + lax.axis_index("c")
        wbase = wid * ew
        qbufs = (qbuf0, qbuf1)
        kvbufs = (kvbuf0, kvbuf1)
        qsems = (qs0, qs1)
        ksems = (ks0, ks1)
        wsems = (ws0, ws1)

        pltpu.sync_copy(dst_hbm.at[pl.ds(wbase, ew)], dstv)
        pltpu.sync_copy(src_hbm.at[pl.ds(wbase, ew)], srcv)

        def do_pair(i, nbuf):
            gq, gk = [], []
            for b in range(nbuf):
                off = (i + b) * GCH
                gq.append(pltpu.async_copy(
                    q_hbm.at[dstv.at[pl.ds(off, GCH)]], qbufs[b], qsems[b]))
                gk.append(pltpu.async_copy(
                    kvv_hbm.at[srcv.at[pl.ds(off, GCH)]], kvbufs[b], ksems[b]))
            ws = []
            for b in range(nbuf):
                off = (i + b) * GCH
                gq[b].wait()
                gk[b].wait()
                ws.append(pltpu.async_copy(
                    qbufs[b], qi_hbm.at[pl.ds(wbase + off, GCH)], wsems[b]))
                ws.append(pltpu.async_copy(
                    kvbufs[b], kvj_hbm.at[pl.ds(wbase + off, GCH)], wsems[b]))
            for w in ws:
                w.wait()

        @pl.loop(0, even_n, step=2)
        def chunks(i):
            do_pair(i, 2)

        if nch % 2:
            do_pair(even_n, 1)

    return gather_k(dst, src, q, kvv)


def _edge_body(f_ref, qi_ref, kvj_ref, r_ref, dT_ref,
               wdkt_ref, bdk_ref, wdvt_ref, bdv_ref, m_ref, y_ref):
    f = f_ref[...]
    dk = _silu(jnp.dot(f, wdkt_ref[...], preferred_element_type=jnp.float32) + bdk_ref[...])
    dv = _silu(jnp.dot(f, wdvt_ref[...], preferred_element_type=jnp.float32) + bdv_ref[...])
    kj = kvj_ref[:, 0:HC]
    vj = kvj_ref[:, HC:4 * HC]
    t = qi_ref[...] * kj * dk
    attnb = jnp.dot(t, m_ref[...], preferred_element_type=jnp.float32)
    r = r_ref[...]
    cut = 0.5 * (jnp.cos(r * (math.pi / CUTOFF)) + 1.0)
    cut = jnp.where(r < CUTOFF, cut, 0.0)
    act = _silu(attnb) * cut
    vdv = vj * dv
    v1 = vdv[:, 0:HC]
    v2 = vdv[:, HC:2 * HC]
    v3 = vdv[:, 2 * HC:3 * HC]
    y_ref[0] = v1 * act
    for d in range(3):
        y_ref[1 + d] = kvj_ref[:, (4 + d) * HC:(5 + d) * HC] * v2 + v3 * dT_ref[d]


def _edge_compute(f_ij, qi, kvj, r2, dT, wdkt, bdk, wdvt, bdv, m, ne):
    grid = (ne // BE,)
    return pl.pallas_call(
        _edge_body,
        grid=grid,
        in_specs=[
            pl.BlockSpec((BE, NRBF), lambda i: (i, 0)),
            pl.BlockSpec((BE, HC), lambda i: (i, 0)),
            pl.BlockSpec((BE, KVW), lambda i: (i, 0)),
            pl.BlockSpec((BE, 1), lambda i: (i, 0)),
            pl.BlockSpec((3, BE, 1), lambda i: (0, i, 0)),
            pl.BlockSpec((NRBF, HC), lambda i: (0, 0)),
            pl.BlockSpec((1, HC), lambda i: (0, 0)),
            pl.BlockSpec((NRBF, 3 * HC), lambda i: (0, 0)),
            pl.BlockSpec((1, 3 * HC), lambda i: (0, 0)),
            pl.BlockSpec((HC, HC), lambda i: (0, 0)),
        ],
        out_specs=pl.BlockSpec((4, BE, HC), lambda i: (0, i, 0)),
        out_shape=jax.ShapeDtypeStruct((4, ne, HC), jnp.float32),
    )(f_ij, qi, kvj, r2, dT, wdkt, bdk, wdvt, bdv, m)


NHALF = NPAD // 2
ACC_R = 5248
ZSTRIPE = ACC_R // 16
FSTRIPE = NHALF // 16
BCH = 80
IB = 80


def _sc_scatter(y, dst, ne):
    mesh = plsc.VectorSubcoreMesh(core_axis_name="c", subcore_axis_name="s")
    et = ne // 16
    nch = et // BCH
    even_n = nch - (nch % 2)

    @functools.partial(
        pl.kernel,
        out_type=jax.ShapeDtypeStruct((4, NPAD, HC), jnp.float32),
        mesh=mesh,
        scratch_types=[
            pltpu.VMEM_SHARED((ACC_R, HC), jnp.float32),
            pltpu.VMEM((BCH, HC), jnp.float32),
            pltpu.VMEM((BCH, HC), jnp.float32),
            pltpu.VMEM((BCH,), jnp.int32),
            pltpu.VMEM((BCH,), jnp.int32),
            pltpu.VMEM((BCH // IB, IB), jnp.int32),
            pltpu.VMEM((BCH // IB, IB), jnp.int32),
            pltpu.SemaphoreType.DMA,
            pltpu.SemaphoreType.DMA,
        ],
    )
    def scatter_k(y_hbm, dst_hbm, agg_hbm, acc,
                  vbuf0, vbuf1, idxv0, idxv1, idx2v0, idx2v1, sem0, sem1):
        c = lax.axis_index("c")
        t = lax.axis_index("s")
        half_base = c * NHALF
        zero16 = jnp.zeros((16,), jnp.float32)
        vbufs = (vbuf0, vbuf1)
        idxvs = (idxv0, idxv1)
        idx2vs = (idx2v0, idx2v1)
        sems = (sem0, sem1)

        def zero_acc():
            def zrow(rr, carry):
                for j in range(HC // 16):
                    vbuf0[rr, pl.ds(j * 16, 16)] = zero16
                return carry

            lax.fori_loop(0, BCH, zrow, 0)
            done = 0
            while done < ZSTRIPE:
                n = min(BCH, ZSTRIPE - done)
                pltpu.sync_copy(vbuf0.at[pl.ds(0, n)],
                                acc.at[pl.ds(t * ZSTRIPE + done, n)])
                done += n

        def remap(b):
            for g in range(BCH // 16):
                v = idxvs[b][pl.ds(g * 16, 16)]
                local = v - half_base
                keep = (local >= 0) & (local < NHALF)
                gj, gk = divmod(g, IB // 16)
                idx2vs[b][gj, pl.ds(gk * 16, 16)] = jnp.where(
                    keep, local, NHALF + (v & 127))

        def scatter_buf(b):
            for j in range(BCH // IB):
                pltpu.sync_copy(vbufs[b].at[pl.ds(j * IB, IB)],
                                acc.at[idx2vs[b].at[j]], add=True)

        zero_acc()
        plsc.subcore_barrier()

        for slab in range(4):
            def do_pair(i, nbuf, slab=slab):
                cps = []
                for b in range(nbuf):
                    base = t * et + (i + b) * BCH
                    pltpu.sync_copy(dst_hbm.at[pl.ds(base, BCH)], idxvs[b])
                    cps.append(pltpu.async_copy(y_hbm.at[slab, pl.ds(base, BCH)],
                                                vbufs[b], sems[b]))
                for b in range(nbuf):
                    remap(b)
                for b in range(nbuf):
                    cps[b].wait()
                    scatter_buf(b)

            @pl.loop(0, even_n, step=2)
            def chunks(i):
                do_pair(i, 2)

            if nch % 2:
                do_pair(even_n, 1)

            plsc.subcore_barrier()
            pltpu.sync_copy(
                acc.at[pl.ds(t * FSTRIPE, FSTRIPE)],
                agg_hbm.at[slab, pl.ds(pl.multiple_of(half_base + t * FSTRIPE, 8),
                                       FSTRIPE)])
            if slab < 3:
                zero_acc()
            plsc.subcore_barrier()

    return scatter_k(y, dst)


def _final_body(agg1_ref, agg2_ref, vp_ref, wot_ref, bo_ref, dx_ref, dvec_ref):
    xa = agg1_ref[0] + agg2_ref[0]
    o = jnp.dot(xa, wot_ref[...], preferred_element_type=jnp.float32) + bo_ref[...]
    o1 = o[:, 0:HC]
    o2 = o[:, HC:2 * HC]
    o3 = o[:, 2 * HC:3 * HC]
    vec_dot = (vp_ref[:, 0, 0:HC] * vp_ref[:, 0, HC:2 * HC]
               + vp_ref[:, 1, 0:HC] * vp_ref[:, 1, HC:2 * HC]
               + vp_ref[:, 2, 0:HC] * vp_ref[:, 2, HC:2 * HC])
    dx_ref[...] = vec_dot * o2 + o3
    for d in range(3):
        dvec_ref[:, d, :] = (vp_ref[:, d, 2 * HC:3 * HC] * o1
                             + agg1_ref[1 + d] + agg2_ref[1 + d])


def _final(agg1, agg2, vp, wot, bo):
    grid = (N // BN,)
    return pl.pallas_call(
        _final_body,
        grid=grid,
        in_specs=[
            pl.BlockSpec((4, BN, HC), lambda i: (0, i, 0)),
            pl.BlockSpec((4, BN, HC), lambda i: (0, i, 0)),
            pl.BlockSpec((BN, 3, 3 * HC), lambda i: (i, 0, 0)),
            pl.BlockSpec((HC, 3 * HC), lambda i: (0, 0)),
            pl.BlockSpec((1, 3 * HC), lambda i: (0, 0)),
        ],
        out_specs=[
            pl.BlockSpec((BN, HC), lambda i: (i, 0)),
            pl.BlockSpec((BN, 3, HC), lambda i: (i, 0, 0)),
        ],
        out_shape=[
            jax.ShapeDtypeStruct((N, HC), jnp.float32),
            jax.ShapeDtypeStruct((N, 3, HC), jnp.float32),
        ],
    )(agg1, agg2, vp, wot, bo)


_s, _h, _j = np.meshgrid(np.arange(3), np.arange(NH), np.arange(HD), indexing="ij")
_PERM = (_h * 3 * HD + _s * HD + _j).reshape(-1)
_M_HEAD = np.kron(np.eye(NH, dtype=np.float32),
                  np.ones((HD, HD), np.float32))


def kernel(x, vec, edge_index, r_ij, f_ij, d_ij, ln_w, ln_b, Wq, bq, Wk, bk,
           Wv, bv, Wo, bo, Wvec, Wdk, bdk, Wdv, bdv):
    src = edge_index[0]
    dst = edge_index[1]

    wqt = Wq.T
    wkt = Wk.T
    wvt = Wv[_PERM].T
    bv_p = bv[_PERM].reshape(1, 3 * HC)
    wvect = Wvec.T
    wdkt = Wdk.T
    wdvt = Wdv[_PERM].T
    bdv_p = bdv[_PERM].reshape(1, 3 * HC)
    wot = Wo.T

    q, kvv, vp = _node_prep(
        x, vec, ln_w.reshape(1, HC), ln_b.reshape(1, HC),
        wqt, bq.reshape(1, HC), wkt, bk.reshape(1, HC), wvt, bv_p, wvect)

    EH = E // 2
    r2 = r_ij.reshape(E, 1)
    dT3 = d_ij.T.reshape(3, E, 1)
    bdk2 = bdk.reshape(1, HC)

    halves = []
    for h in range(2):
        e0 = h * EH
        halves.append((dst[e0:e0 + EH], src[e0:e0 + EH], f_ij[e0:e0 + EH],
                       r2[e0:e0 + EH], dT3[:, e0:e0 + EH]))

    g1 = _sc_gather(halves[0][0], halves[0][1], q, kvv, EH)
    g2 = _sc_gather(halves[1][0], halves[1][1], q, kvv, EH)
    y1 = _edge_compute(halves[0][2], g1[0], g1[1], halves[0][3], halves[0][4],
                       wdkt, bdk2, wdvt, bdv_p, _M_HEAD, EH)
    agg1 = _sc_scatter(y1, halves[0][0], EH)
    y2 = _edge_compute(halves[1][2], g2[0], g2[1], halves[1][3], halves[1][4],
                       wdkt, bdk2, wdvt, bdv_p, _M_HEAD, EH)
    agg2 = _sc_scatter(y2, halves[1][0], EH)

    dx, dvec = _final(agg1, agg2, vp, wot, bo.reshape(1, 3 * HC))
    return dx, dvec


# speedup vs baseline: 30.2560x; 1.1197x over previous
import functools
import math

import numpy as np
import jax
import jax.numpy as jnp
from jax import lax
from jax.experimental import pallas as pl
from jax.experimental.pallas import tpu as pltpu
from jax.experimental.pallas import tpu_sc as plsc

N = 10000
E = 320000
HC = 128
NH = 8
HD = 16
NRBF = 64
CUTOFF = 5.0
EPS = 1e-5
NPAD = 10240

KVW = 4 * HC

BN = 1000
BE = 1280

NW = 32
EW = E // NW
GCH = 40
ET = E // 16


def _silu(x):
    return x * jax.nn.sigmoid(x)


def _node_prep_body(x_ref, vec_ref, lnw_ref, lnb_ref, wqt_ref, bq_ref,
                    wkt_ref, bk_ref, wvt_ref, bv_ref, wvect_ref,
                    q_ref, kvv_ref, vp_ref):
    x = x_ref[...]
    mu = jnp.mean(x, axis=-1, keepdims=True)
    xc = x - mu
    var = jnp.mean(xc * xc, axis=-1, keepdims=True)
    xn = xc * lax.rsqrt(var + EPS) * lnw_ref[...] + lnb_ref[...]
    q_ref[...] = jnp.dot(xn, wqt_ref[...], preferred_element_type=jnp.float32) + bq_ref[...]
    kj = jnp.dot(xn, wkt_ref[...], preferred_element_type=jnp.float32) + bk_ref[...]
    vv = jnp.dot(xn, wvt_ref[...], preferred_element_type=jnp.float32) + bv_ref[...]

    def _hi16(a):
        bi = lax.bitcast_convert_type(a, jnp.int32)
        return (bi + 0x8000) & jnp.int32(-65536)

    def _lo16(a):
        return lax.shift_right_logical(_hi16(a), 16)

    los = [kj, vv[:, 0:HC], vv[:, HC:2 * HC], vv[:, 2 * HC:3 * HC]]
    his = [vec_ref[:, 0, :], vec_ref[:, 1, :], vec_ref[:, 2, :],
           jnp.zeros((BN, HC), jnp.float32)]
    for m in range(4):
        kvv_ref[:, m * HC:(m + 1) * HC] = _lo16(los[m]) | _hi16(his[m])
    for d in range(3):
        vp_ref[:, d, :] = jnp.dot(vec_ref[:, d, :], wvect_ref[...],
                                  preferred_element_type=jnp.float32)


def _node_prep(x, vec, lnw, lnb, wqt, bq, wkt, bk, wvt, bv, wvect):
    grid = (N // BN,)
    return pl.pallas_call(
        _node_prep_body,
        grid=grid,
        in_specs=[
            pl.BlockSpec((BN, HC), lambda i: (i, 0)),
            pl.BlockSpec((BN, 3, HC), lambda i: (i, 0, 0)),
            pl.BlockSpec((1, HC), lambda i: (0, 0)),
            pl.BlockSpec((1, HC), lambda i: (0, 0)),
            pl.BlockSpec((HC, HC), lambda i: (0, 0)),
            pl.BlockSpec((1, HC), lambda i: (0, 0)),
            pl.BlockSpec((HC, HC), lambda i: (0, 0)),
            pl.BlockSpec((1, HC), lambda i: (0, 0)),
            pl.BlockSpec((HC, 3 * HC), lambda i: (0, 0)),
            pl.BlockSpec((1, 3 * HC), lambda i: (0, 0)),
            pl.BlockSpec((HC, 3 * HC), lambda i: (0, 0)),
        ],
        out_specs=[
            pl.BlockSpec((BN, HC), lambda i: (i, 0)),
            pl.BlockSpec((BN, KVW), lambda i: (i, 0)),
            pl.BlockSpec((BN, 3, 3 * HC), lambda i: (i, 0, 0)),
        ],
        out_shape=[
            jax.ShapeDtypeStruct((N, HC), jnp.float32),
            jax.ShapeDtypeStruct((N, KVW), jnp.int32),
            jax.ShapeDtypeStruct((N, 3, 3 * HC), jnp.float32),
        ],
    )(x, vec, lnw, lnb, wqt, bq, wkt, bk, wvt, bv, wvect)


def _sc_gather(dst, src, q, kvv, ne):
    mesh = plsc.VectorSubcoreMesh(core_axis_name="c", subcore_axis_name="s")
    ew = ne // NW
    nch = ew // GCH
    even_n = nch - (nch % 2)

    @functools.partial(
        pl.kernel,
        out_type=[
            jax.ShapeDtypeStruct((ne, HC), jnp.float32),
            jax.ShapeDtypeStruct((ne, KVW), jnp.int32),
        ],
        mesh=mesh,
        scratch_types=[
            pltpu.VMEM((ew,), jnp.int32),
            pltpu.VMEM((ew,), jnp.int32),
            pltpu.VMEM((GCH, HC), jnp.float32),
            pltpu.VMEM((GCH, HC), jnp.float32),
            pltpu.VMEM((GCH, KVW), jnp.int32),
            pltpu.VMEM((GCH, KVW), jnp.int32),
            pltpu.SemaphoreType.DMA,
            pltpu.SemaphoreType.DMA,
            pltpu.SemaphoreType.DMA,
            pltpu.SemaphoreType.DMA,
            pltpu.SemaphoreType.DMA,
            pltpu.SemaphoreType.DMA,
        ],
    )
    def gather_k(dst_hbm, src_hbm, q_hbm, kvv_hbm, qi_hbm, kvj_hbm,
                 dstv, srcv, qbuf0, qbuf1, kvbuf0, kvbuf1,
                 qs0, qs1, ks0, ks1, ws0, ws1):
        wid = lax.axis_index("s") * 2 + lax.axis_index("c")
        wbase = wid * ew
        qbufs = (qbuf0, qbuf1)
        kvbufs = (kvbuf0, kvbuf1)
        qsems = (qs0, qs1)
        ksems = (ks0, ks1)
        wsems = (ws0, ws1)

        pltpu.sync_copy(dst_hbm.at[pl.ds(wbase, ew)], dstv)
        pltpu.sync_copy(src_hbm.at[pl.ds(wbase, ew)], srcv)

        def do_pair(i, nbuf):
            gq, gk = [], []
            for b in range(nbuf):
                off = (i + b) * GCH
                gq.append(pltpu.async_copy(
                    q_hbm.at[dstv.at[pl.ds(off, GCH)]], qbufs[b], qsems[b]))
                gk.append(pltpu.async_copy(
                    kvv_hbm.at[srcv.at[pl.ds(off, GCH)]], kvbufs[b], ksems[b]))
            ws = []
            for b in range(nbuf):
                off = (i + b) * GCH
                gq[b].wait()
                gk[b].wait()
                ws.append(pltpu.async_copy(
                    qbufs[b], qi_hbm.at[pl.ds(wbase + off, GCH)], wsems[b]))
                ws.append(pltpu.async_copy(
                    kvbufs[b], kvj_hbm.at[pl.ds(wbase + off, GCH)], wsems[b]))
            for w in ws:
                w.wait()

        @pl.loop(0, even_n, step=2)
        def chunks(i):
            do_pair(i, 2)

        if nch % 2:
            do_pair(even_n, 1)

    return gather_k(dst, src, q, kvv)


def _edge_body(f_ref, qi_ref, kvj_ref, r_ref, dT_ref,
               wdkt_ref, bdk_ref, wdvt_ref, bdv_ref, m_ref, y_ref):
    f = f_ref[...]
    dk = _silu(jnp.dot(f, wdkt_ref[...], preferred_element_type=jnp.float32) + bdk_ref[...])
    dv = _silu(jnp.dot(f, wdvt_ref[...], preferred_element_type=jnp.float32) + bdv_ref[...])
    xw = kvj_ref[...]
    kvf = lax.bitcast_convert_type(lax.shift_left(xw, 16), jnp.float32)
    vecf = lax.bitcast_convert_type(xw & jnp.int32(-65536), jnp.float32)
    kj = kvf[:, 0:HC]
    vj = kvf[:, HC:4 * HC]
    t = qi_ref[...] * kj * dk
    attnb = jnp.dot(t, m_ref[...], preferred_element_type=jnp.float32)
    r = r_ref[...]
    cut = 0.5 * (jnp.cos(r * (math.pi / CUTOFF)) + 1.0)
    cut = jnp.where(r < CUTOFF, cut, 0.0)
    act = _silu(attnb) * cut
    vdv = vj * dv
    v1 = vdv[:, 0:HC]
    v2 = vdv[:, HC:2 * HC]
    v3 = vdv[:, 2 * HC:3 * HC]
    y_ref[0] = v1 * act
    for d in range(3):
        y_ref[1 + d] = vecf[:, d * HC:(d + 1) * HC] * v2 + v3 * dT_ref[d]


def _edge_compute(f_ij, qi, kvj, r2, dT, wdkt, bdk, wdvt, bdv, m, ne):
    grid = (ne // BE,)
    return pl.pallas_call(
        _edge_body,
        grid=grid,
        in_specs=[
            pl.BlockSpec((BE, NRBF), lambda i: (i, 0)),
            pl.BlockSpec((BE, HC), lambda i: (i, 0)),
            pl.BlockSpec((BE, KVW), lambda i: (i, 0)),
            pl.BlockSpec((BE, 1), lambda i: (i, 0)),
            pl.BlockSpec((3, BE, 1), lambda i: (0, i, 0)),
            pl.BlockSpec((NRBF, HC), lambda i: (0, 0)),
            pl.BlockSpec((1, HC), lambda i: (0, 0)),
            pl.BlockSpec((NRBF, 3 * HC), lambda i: (0, 0)),
            pl.BlockSpec((1, 3 * HC), lambda i: (0, 0)),
            pl.BlockSpec((HC, HC), lambda i: (0, 0)),
        ],
        out_specs=pl.BlockSpec((4, BE, HC), lambda i: (0, i, 0)),
        out_shape=jax.ShapeDtypeStruct((4, ne, HC), jnp.float32),
    )(f_ij, qi, kvj, r2, dT, wdkt, bdk, wdvt, bdv, m)


NHALF = NPAD // 2
ACC_R = 5248
ZSTRIPE = ACC_R // 16
FSTRIPE = NHALF // 16
BCH = 80
IB = 80


def _sc_scatter(y, dst, ne):
    mesh = plsc.VectorSubcoreMesh(core_axis_name="c", subcore_axis_name="s")
    et = ne // 16
    nch = et // BCH
    even_n = nch - (nch % 2)

    @functools.partial(
        pl.kernel,
        out_type=jax.ShapeDtypeStruct((4, NPAD, HC), jnp.float32),
        mesh=mesh,
        scratch_types=[
            pltpu.VMEM_SHARED((ACC_R, HC), jnp.float32),
            pltpu.VMEM((BCH, HC), jnp.float32),
            pltpu.VMEM((BCH, HC), jnp.float32),
            pltpu.VMEM((BCH,), jnp.int32),
            pltpu.VMEM((BCH,), jnp.int32),
            pltpu.VMEM((BCH // IB, IB), jnp.int32),
            pltpu.VMEM((BCH // IB, IB), jnp.int32),
            pltpu.SemaphoreType.DMA,
            pltpu.SemaphoreType.DMA,
        ],
    )
    def scatter_k(y_hbm, dst_hbm, agg_hbm, acc,
                  vbuf0, vbuf1, idxv0, idxv1, idx2v0, idx2v1, sem0, sem1):
        c = lax.axis_index("c")
        t = lax.axis_index("s")
        half_base = c * NHALF
        zero16 = jnp.zeros((16,), jnp.float32)
        vbufs = (vbuf0, vbuf1)
        idxvs = (idxv0, idxv1)
        idx2vs = (idx2v0, idx2v1)
        sems = (sem0, sem1)

        def zero_acc():
            def zrow(rr, carry):
                for j in range(HC // 16):
                    vbuf0[rr, pl.ds(j * 16, 16)] = zero16
                return carry

            lax.fori_loop(0, BCH, zrow, 0)
            done = 0
            while done < ZSTRIPE:
                n = min(BCH, ZSTRIPE - done)
                pltpu.sync_copy(vbuf0.at[pl.ds(0, n)],
                                acc.at[pl.ds(t * ZSTRIPE + done, n)])
                done += n

        def remap(b):
            for g in range(BCH // 16):
                v = idxvs[b][pl.ds(g * 16, 16)]
                local = v - half_base
                keep = (local >= 0) & (local < NHALF)
                gj, gk = divmod(g, IB // 16)
                idx2vs[b][gj, pl.ds(gk * 16, 16)] = jnp.where(
                    keep, local, NHALF + (v & 127))

        def scatter_buf(b):
            for j in range(BCH // IB):
                pltpu.sync_copy(vbufs[b].at[pl.ds(j * IB, IB)],
                                acc.at[idx2vs[b].at[j]], add=True)

        zero_acc()
        plsc.subcore_barrier()

        for slab in range(4):
            def do_pair(i, nbuf, slab=slab):
                cps = []
                for b in range(nbuf):
                    base = t * et + (i + b) * BCH
                    pltpu.sync_copy(dst_hbm.at[pl.ds(base, BCH)], idxvs[b])
                    cps.append(pltpu.async_copy(y_hbm.at[slab, pl.ds(base, BCH)],
                                                vbufs[b], sems[b]))
                for b in range(nbuf):
                    remap(b)
                for b in range(nbuf):
                    cps[b].wait()
                    scatter_buf(b)

            @pl.loop(0, even_n, step=2)
            def chunks(i):
                do_pair(i, 2)

            if nch % 2:
                do_pair(even_n, 1)

            plsc.subcore_barrier()
            pltpu.sync_copy(
                acc.at[pl.ds(t * FSTRIPE, FSTRIPE)],
                agg_hbm.at[slab, pl.ds(pl.multiple_of(half_base + t * FSTRIPE, 8),
                                       FSTRIPE)])
            if slab < 3:
                zero_acc()
            plsc.subcore_barrier()

    return scatter_k(y, dst)


def _final_body(agg1_ref, agg2_ref, vp_ref, wot_ref, bo_ref, dx_ref, dvec_ref):
    xa = agg1_ref[0] + agg2_ref[0]
    o = jnp.dot(xa, wot_ref[...], preferred_element_type=jnp.float32) + bo_ref[...]
    o1 = o[:, 0:HC]
    o2 = o[:, HC:2 * HC]
    o3 = o[:, 2 * HC:3 * HC]
    vec_dot = (vp_ref[:, 0, 0:HC] * vp_ref[:, 0, HC:2 * HC]
               + vp_ref[:, 1, 0:HC] * vp_ref[:, 1, HC:2 * HC]
               + vp_ref[:, 2, 0:HC] * vp_ref[:, 2, HC:2 * HC])
    dx_ref[...] = vec_dot * o2 + o3
    for d in range(3):
        dvec_ref[:, d, :] = (vp_ref[:, d, 2 * HC:3 * HC] * o1
                             + agg1_ref[1 + d] + agg2_ref[1 + d])


def _final(agg1, agg2, vp, wot, bo):
    grid = (N // BN,)
    return pl.pallas_call(
        _final_body,
        grid=grid,
        in_specs=[
            pl.BlockSpec((4, BN, HC), lambda i: (0, i, 0)),
            pl.BlockSpec((4, BN, HC), lambda i: (0, i, 0)),
            pl.BlockSpec((BN, 3, 3 * HC), lambda i: (i, 0, 0)),
            pl.BlockSpec((HC, 3 * HC), lambda i: (0, 0)),
            pl.BlockSpec((1, 3 * HC), lambda i: (0, 0)),
        ],
        out_specs=[
            pl.BlockSpec((BN, HC), lambda i: (i, 0)),
            pl.BlockSpec((BN, 3, HC), lambda i: (i, 0, 0)),
        ],
        out_shape=[
            jax.ShapeDtypeStruct((N, HC), jnp.float32),
            jax.ShapeDtypeStruct((N, 3, HC), jnp.float32),
        ],
    )(agg1, agg2, vp, wot, bo)


_s, _h, _j = np.meshgrid(np.arange(3), np.arange(NH), np.arange(HD), indexing="ij")
_PERM = (_h * 3 * HD + _s * HD + _j).reshape(-1)
_M_HEAD = np.kron(np.eye(NH, dtype=np.float32),
                  np.ones((HD, HD), np.float32))


def kernel(x, vec, edge_index, r_ij, f_ij, d_ij, ln_w, ln_b, Wq, bq, Wk, bk,
           Wv, bv, Wo, bo, Wvec, Wdk, bdk, Wdv, bdv):
    src = edge_index[0]
    dst = edge_index[1]

    wqt = Wq.T
    wkt = Wk.T
    wvt = Wv[_PERM].T
    bv_p = bv[_PERM].reshape(1, 3 * HC)
    wvect = Wvec.T
    wdkt = Wdk.T
    wdvt = Wdv[_PERM].T
    bdv_p = bdv[_PERM].reshape(1, 3 * HC)
    wot = Wo.T

    q, kvv, vp = _node_prep(
        x, vec, ln_w.reshape(1, HC), ln_b.reshape(1, HC),
        wqt, bq.reshape(1, HC), wkt, bk.reshape(1, HC), wvt, bv_p, wvect)

    EH = E // 2
    r2 = r_ij.reshape(E, 1)
    dT3 = d_ij.T.reshape(3, E, 1)
    bdk2 = bdk.reshape(1, HC)

    halves = []
    for h in range(2):
        e0 = h * EH
        halves.append((dst[e0:e0 + EH], src[e0:e0 + EH], f_ij[e0:e0 + EH],
                       r2[e0:e0 + EH], dT3[:, e0:e0 + EH]))

    g1 = _sc_gather(halves[0][0], halves[0][1], q, kvv, EH)
    g2 = _sc_gather(halves[1][0], halves[1][1], q, kvv, EH)
    y1 = _edge_compute(halves[0][2], g1[0], g1[1], halves[0][3], halves[0][4],
                       wdkt, bdk2, wdvt, bdv_p, _M_HEAD, EH)
    agg1 = _sc_scatter(y1, halves[0][0], EH)
    y2 = _edge_compute(halves[1][2], g2[0], g2[1], halves[1][3], halves[1][4],
                       wdkt, bdk2, wdvt, bdv_p, _M_HEAD, EH)
    agg2 = _sc_scatter(y2, halves[1][0], EH)

    dx, dvec = _final(agg1, agg2, vp, wot, bo.reshape(1, 3 * HC))
    return dx, dvec


# speedup vs baseline: 31.4654x; 1.0400x over previous
import functools
import math

import numpy as np
import jax
import jax.numpy as jnp
from jax import lax
from jax.experimental import pallas as pl
from jax.experimental.pallas import tpu as pltpu
from jax.experimental.pallas import tpu_sc as plsc

N = 10000
E = 320000
HC = 128
NH = 8
HD = 16
NRBF = 64
CUTOFF = 5.0
EPS = 1e-5
NPAD = 10240

KVW = 4 * HC

BN = 1000
BE = 1280

NW = 32
EW = E // NW
GCH = 40
ET = E // 16


def _silu(x):
    return x * jax.nn.sigmoid(x)


def _node_prep_body(x_ref, vec_ref, lnw_ref, lnb_ref, wqt_ref, bq_ref,
                    wkt_ref, bk_ref, wvt_ref, bv_ref, wvect_ref,
                    q_ref, kvv_ref, vp_ref):
    x = x_ref[...]
    mu = jnp.mean(x, axis=-1, keepdims=True)
    xc = x - mu
    var = jnp.mean(xc * xc, axis=-1, keepdims=True)
    xn = xc * lax.rsqrt(var + EPS) * lnw_ref[...] + lnb_ref[...]
    q_ref[...] = jnp.dot(xn, wqt_ref[...], preferred_element_type=jnp.float32) + bq_ref[...]
    kj = jnp.dot(xn, wkt_ref[...], preferred_element_type=jnp.float32) + bk_ref[...]
    vv = jnp.dot(xn, wvt_ref[...], preferred_element_type=jnp.float32) + bv_ref[...]

    def _hi16(a):
        bi = lax.bitcast_convert_type(a, jnp.int32)
        return (bi + 0x8000) & jnp.int32(-65536)

    def _lo16(a):
        return lax.shift_right_logical(_hi16(a), 16)

    los = [kj, vv[:, 0:HC], vv[:, HC:2 * HC], vv[:, 2 * HC:3 * HC]]
    his = [vec_ref[:, 0, :], vec_ref[:, 1, :], vec_ref[:, 2, :],
           jnp.zeros((BN, HC), jnp.float32)]
    for m in range(4):
        kvv_ref[:, m * HC:(m + 1) * HC] = _lo16(los[m]) | _hi16(his[m])
    for d in range(3):
        vp_ref[:, d, :] = jnp.dot(vec_ref[:, d, :], wvect_ref[...],
                                  preferred_element_type=jnp.float32)


def _node_prep(x, vec, lnw, lnb, wqt, bq, wkt, bk, wvt, bv, wvect):
    grid = (N // BN,)
    return pl.pallas_call(
        _node_prep_body,
        grid=grid,
        in_specs=[
            pl.BlockSpec((BN, HC), lambda i: (i, 0)),
            pl.BlockSpec((BN, 3, HC), lambda i: (i, 0, 0)),
            pl.BlockSpec((1, HC), lambda i: (0, 0)),
            pl.BlockSpec((1, HC), lambda i: (0, 0)),
            pl.BlockSpec((HC, HC), lambda i: (0, 0)),
            pl.BlockSpec((1, HC), lambda i: (0, 0)),
            pl.BlockSpec((HC, HC), lambda i: (0, 0)),
            pl.BlockSpec((1, HC), lambda i: (0, 0)),
            pl.BlockSpec((HC, 3 * HC), lambda i: (0, 0)),
            pl.BlockSpec((1, 3 * HC), lambda i: (0, 0)),
            pl.BlockSpec((HC, 3 * HC), lambda i: (0, 0)),
        ],
        out_specs=[
            pl.BlockSpec((BN, HC), lambda i: (i, 0)),
            pl.BlockSpec((BN, KVW), lambda i: (i, 0)),
            pl.BlockSpec((BN, 3, 3 * HC), lambda i: (i, 0, 0)),
        ],
        out_shape=[
            jax.ShapeDtypeStruct((N, HC), jnp.float32),
            jax.ShapeDtypeStruct((N, KVW), jnp.int32),
            jax.ShapeDtypeStruct((N, 3, 3 * HC), jnp.float32),
        ],
    )(x, vec, lnw, lnb, wqt, bq, wkt, bk, wvt, bv, wvect)


def _sc_gather(dst, src, q, kvv, ne):
    mesh = plsc.VectorSubcoreMesh(core_axis_name="c", subcore_axis_name="s")
    ew = ne // NW
    nch = ew // GCH
    even_n = nch - (nch % 2)

    @functools.partial(
        pl.kernel,
        out_type=[
            jax.ShapeDtypeStruct((ne, HC), jnp.float32),
            jax.ShapeDtypeStruct((ne, KVW), jnp.int32),
        ],
        mesh=mesh,
        scratch_types=[
            pltpu.VMEM((ew,), jnp.int32),
            pltpu.VMEM((ew,), jnp.int32),
            pltpu.VMEM((GCH, HC), jnp.float32),
            pltpu.VMEM((GCH, HC), jnp.float32),
            pltpu.VMEM((GCH, KVW), jnp.int32),
            pltpu.VMEM((GCH, KVW), jnp.int32),
            pltpu.SemaphoreType.DMA,
            pltpu.SemaphoreType.DMA,
            pltpu.SemaphoreType.DMA,
            pltpu.SemaphoreType.DMA,
            pltpu.SemaphoreType.DMA,
            pltpu.SemaphoreType.DMA,
        ],
    )
    def gather_k(dst_hbm, src_hbm, q_hbm, kvv_hbm, qi_hbm, kvj_hbm,
                 dstv, srcv, qbuf0, qbuf1, kvbuf0, kvbuf1,
                 qs0, qs1, ks0, ks1, ws0, ws1):
        wid = lax.axis_index("s") * 2 + lax.axis_index("c")
        wbase = wid * ew
        qbufs = (qbuf0, qbuf1)
        kvbufs = (kvbuf0, kvbuf1)
        qsems = (qs0, qs1)
        ksems = (ks0, ks1)
        wsems = (ws0, ws1)

        pltpu.sync_copy(dst_hbm.at[pl.ds(wbase, ew)], dstv)
        pltpu.sync_copy(src_hbm.at[pl.ds(wbase, ew)], srcv)

        def do_pair(i, nbuf):
            gq, gk = [], []
            for b in range(nbuf):
                off = (i + b) * GCH
                gq.append(pltpu.async_copy(
                    q_hbm.at[dstv.at[pl.ds(off, GCH)]], qbufs[b], qsems[b]))
                gk.append(pltpu.async_copy(
                    kvv_hbm.at[srcv.at[pl.ds(off, GCH)]], kvbufs[b], ksems[b]))
            ws = []
            for b in range(nbuf):
                off = (i + b) * GCH
                gq[b].wait()
                gk[b].wait()
                ws.append(pltpu.async_copy(
                    qbufs[b], qi_hbm.at[pl.ds(wbase + off, GCH)], wsems[b]))
                ws.append(pltpu.async_copy(
                    kvbufs[b], kvj_hbm.at[pl.ds(wbase + off, GCH)], wsems[b]))
            for w in ws:
                w.wait()

        @pl.loop(0, even_n, step=2)
        def chunks(i):
            do_pair(i, 2)

        if nch % 2:
            do_pair(even_n, 1)

    return gather_k(dst, src, q, kvv)


def _edge_body(f_ref, qi_ref, kvj_ref, r_ref, dT_ref,
               wdkt_ref, bdk_ref, wdvt_ref, bdv_ref, m_ref, y_ref):
    f = f_ref[...]
    dk = _silu(jnp.dot(f, wdkt_ref[...], preferred_element_type=jnp.float32) + bdk_ref[...])
    dv = _silu(jnp.dot(f, wdvt_ref[...], preferred_element_type=jnp.float32) + bdv_ref[...])
    xw = kvj_ref[...]
    kvf = lax.bitcast_convert_type(lax.shift_left(xw, 16), jnp.float32)
    vecf = lax.bitcast_convert_type(xw & jnp.int32(-65536), jnp.float32)
    kj = kvf[:, 0:HC]
    vj = kvf[:, HC:4 * HC]
    t = qi_ref[...] * kj * dk
    attnb = jnp.dot(t, m_ref[...], preferred_element_type=jnp.float32)
    r = r_ref[...]
    cut = 0.5 * (jnp.cos(r * (math.pi / CUTOFF)) + 1.0)
    cut = jnp.where(r < CUTOFF, cut, 0.0)
    act = _silu(attnb) * cut
    vdv = vj * dv
    v1 = vdv[:, 0:HC]
    v2 = vdv[:, HC:2 * HC]
    v3 = vdv[:, 2 * HC:3 * HC]
    y_ref[0] = v1 * act
    for d in range(3):
        y_ref[1 + d] = vecf[:, d * HC:(d + 1) * HC] * v2 + v3 * dT_ref[d]


def _edge_compute(f_ij, qi, kvj, r2, dT, wdkt, bdk, wdvt, bdv, m, ne):
    grid = (ne // BE,)
    return pl.pallas_call(
        _edge_body,
        grid=grid,
        in_specs=[
            pl.BlockSpec((BE, NRBF), lambda i: (i, 0)),
            pl.BlockSpec((BE, HC), lambda i: (i, 0)),
            pl.BlockSpec((BE, KVW), lambda i: (i, 0)),
            pl.BlockSpec((BE, 1), lambda i: (i, 0)),
            pl.BlockSpec((3, BE, 1), lambda i: (0, i, 0)),
            pl.BlockSpec((NRBF, HC), lambda i: (0, 0)),
            pl.BlockSpec((1, HC), lambda i: (0, 0)),
            pl.BlockSpec((NRBF, 3 * HC), lambda i: (0, 0)),
            pl.BlockSpec((1, 3 * HC), lambda i: (0, 0)),
            pl.BlockSpec((HC, HC), lambda i: (0, 0)),
        ],
        out_specs=pl.BlockSpec((4, BE, HC), lambda i: (0, i, 0)),
        out_shape=jax.ShapeDtypeStruct((4, ne, HC), jnp.float32),
    )(f_ij, qi, kvj, r2, dT, wdkt, bdk, wdvt, bdv, m)


NHALF = NPAD // 2
ACC_R = 5248
ZSTRIPE = ACC_R // 16
FSTRIPE = NHALF // 16
BCH = 80
IB = 80


def _sc_scatter(y, dst, ne):
    mesh = plsc.VectorSubcoreMesh(core_axis_name="c", subcore_axis_name="s")
    et = ne // 16
    nch = et // BCH
    even_n = nch - (nch % 2)

    @functools.partial(
        pl.kernel,
        out_type=jax.ShapeDtypeStruct((4, NPAD, HC), jnp.float32),
        mesh=mesh,
        scratch_types=[
            pltpu.VMEM_SHARED((ACC_R, HC), jnp.float32),
            pltpu.VMEM((BCH, HC), jnp.float32),
            pltpu.VMEM((BCH, HC), jnp.float32),
            pltpu.VMEM((ne // 16,), jnp.int32),
            pltpu.VMEM((ne // 16 // BCH, BCH), jnp.int32),
            pltpu.SemaphoreType.DMA,
            pltpu.SemaphoreType.DMA,
        ],
    )
    def scatter_k(y_hbm, dst_hbm, agg_hbm, acc,
                  vbuf0, vbuf1, dstv, idx2all, sem0, sem1):
        c = lax.axis_index("c")
        t = lax.axis_index("s")
        half_base = c * NHALF
        zero16 = jnp.zeros((16,), jnp.float32)
        vbufs = (vbuf0, vbuf1)
        sems = (sem0, sem1)

        def zero_acc():
            def zrow(rr, carry):
                for j in range(HC // 16):
                    vbuf0[rr, pl.ds(j * 16, 16)] = zero16
                return carry

            lax.fori_loop(0, BCH, zrow, 0)
            done = 0
            while done < ZSTRIPE:
                n = min(BCH, ZSTRIPE - done)
                pltpu.sync_copy(vbuf0.at[pl.ds(0, n)],
                                acc.at[pl.ds(t * ZSTRIPE + done, n)])
                done += n

        pltpu.sync_copy(dst_hbm.at[pl.ds(t * et, et)], dstv)

        def remap_chunk(ch, carry):
            for g in range(BCH // 16):
                v = dstv[pl.ds(ch * BCH + g * 16, 16)]
                local = v - half_base
                keep = (local >= 0) & (local < NHALF)
                idx2all[ch, pl.ds(g * 16, 16)] = jnp.where(
                    keep, local, NHALF + (v & 127))
            return carry

        lax.fori_loop(0, nch, remap_chunk, 0)

        zero_acc()
        plsc.subcore_barrier()

        for slab in range(4):
            def do_pair(i, nbuf, slab=slab):
                cps = []
                for b in range(nbuf):
                    base = t * et + (i + b) * BCH
                    cps.append(pltpu.async_copy(y_hbm.at[slab, pl.ds(base, BCH)],
                                                vbufs[b], sems[b]))
                for b in range(nbuf):
                    cps[b].wait()
                    pltpu.sync_copy(vbufs[b], acc.at[idx2all.at[i + b]],
                                    add=True)

            @pl.loop(0, even_n, step=2)
            def chunks(i):
                do_pair(i, 2)

            if nch % 2:
                do_pair(even_n, 1)

            plsc.subcore_barrier()
            pltpu.sync_copy(
                acc.at[pl.ds(t * FSTRIPE, FSTRIPE)],
                agg_hbm.at[slab, pl.ds(pl.multiple_of(half_base + t * FSTRIPE, 8),
                                       FSTRIPE)])
            if slab < 3:
                zero_acc()
            plsc.subcore_barrier()

    return scatter_k(y, dst)


def _final_body(agg1_ref, agg2_ref, vp_ref, wot_ref, bo_ref, dx_ref, dvec_ref):
    xa = agg1_ref[0] + agg2_ref[0]
    o = jnp.dot(xa, wot_ref[...], preferred_element_type=jnp.float32) + bo_ref[...]
    o1 = o[:, 0:HC]
    o2 = o[:, HC:2 * HC]
    o3 = o[:, 2 * HC:3 * HC]
    vec_dot = (vp_ref[:, 0, 0:HC] * vp_ref[:, 0, HC:2 * HC]
               + vp_ref[:, 1, 0:HC] * vp_ref[:, 1, HC:2 * HC]
               + vp_ref[:, 2, 0:HC] * vp_ref[:, 2, HC:2 * HC])
    dx_ref[...] = vec_dot * o2 + o3
    for d in range(3):
        dvec_ref[:, d, :] = (vp_ref[:, d, 2 * HC:3 * HC] * o1
                             + agg1_ref[1 + d] + agg2_ref[1 + d])


def _final(agg1, agg2, vp, wot, bo):
    grid = (N // BN,)
    return pl.pallas_call(
        _final_body,
        grid=grid,
        in_specs=[
            pl.BlockSpec((4, BN, HC), lambda i: (0, i, 0)),
            pl.BlockSpec((4, BN, HC), lambda i: (0, i, 0)),
            pl.BlockSpec((BN, 3, 3 * HC), lambda i: (i, 0, 0)),
            pl.BlockSpec((HC, 3 * HC), lambda i: (0, 0)),
            pl.BlockSpec((1, 3 * HC), lambda i: (0, 0)),
        ],
        out_specs=[
            pl.BlockSpec((BN, HC), lambda i: (i, 0)),
            pl.BlockSpec((BN, 3, HC), lambda i: (i, 0, 0)),
        ],
        out_shape=[
            jax.ShapeDtypeStruct((N, HC), jnp.float32),
            jax.ShapeDtypeStruct((N, 3, HC), jnp.float32),
        ],
    )(agg1, agg2, vp, wot, bo)


_s, _h, _j = np.meshgrid(np.arange(3), np.arange(NH), np.arange(HD), indexing="ij")
_PERM = (_h * 3 * HD + _s * HD + _j).reshape(-1)
_M_HEAD = np.kron(np.eye(NH, dtype=np.float32),
                  np.ones((HD, HD), np.float32))


def kernel(x, vec, edge_index, r_ij, f_ij, d_ij, ln_w, ln_b, Wq, bq, Wk, bk,
           Wv, bv, Wo, bo, Wvec, Wdk, bdk, Wdv, bdv):
    src = edge_index[0]
    dst = edge_index[1]

    wqt = Wq.T
    wkt = Wk.T
    wvt = Wv[_PERM].T
    bv_p = bv[_PERM].reshape(1, 3 * HC)
    wvect = Wvec.T
    wdkt = Wdk.T
    wdvt = Wdv[_PERM].T
    bdv_p = bdv[_PERM].reshape(1, 3 * HC)
    wot = Wo.T

    q, kvv, vp = _node_prep(
        x, vec, ln_w.reshape(1, HC), ln_b.reshape(1, HC),
        wqt, bq.reshape(1, HC), wkt, bk.reshape(1, HC), wvt, bv_p, wvect)

    EH = E // 2
    r2 = r_ij.reshape(E, 1)
    dT3 = d_ij.T.reshape(3, E, 1)
    bdk2 = bdk.reshape(1, HC)

    halves = []
    for h in range(2):
        e0 = h * EH
        halves.append((dst[e0:e0 + EH], src[e0:e0 + EH], f_ij[e0:e0 + EH],
                       r2[e0:e0 + EH], dT3[:, e0:e0 + EH]))

    g1 = _sc_gather(halves[0][0], halves[0][1], q, kvv, EH)
    g2 = _sc_gather(halves[1][0], halves[1][1], q, kvv, EH)
    y1 = _edge_compute(halves[0][2], g1[0], g1[1], halves[0][3], halves[0][4],
                       wdkt, bdk2, wdvt, bdv_p, _M_HEAD, EH)
    agg1 = _sc_scatter(y1, halves[0][0], EH)
    y2 = _edge_compute(halves[1][2], g2[0], g2[1], halves[1][3], halves[1][4],
                       wdkt, bdk2, wdvt, bdv_p, _M_HEAD, EH)
    agg2 = _sc_scatter(y2, halves[1][0], EH)

    dx, dvec = _final(agg1, agg2, vp, wot, bo.reshape(1, 3 * HC))
    return dx, dvec
